# Initial kernel scaffold; baseline (speedup 1.0000x reference)
import jax
import jax.numpy as jnp
from jax.experimental import pallas as pl


def kernel(word_ids, char_ids, video_features, v_mask, q_mask, word_emb, char_emb, W_emb, b_emb, W_v, b_v, W_enc, b_enc, Wq, Wk, Wv, Wskip, W_start, b_start, W_end, b_end):
    raise NotImplementedError("write your pallas kernel here")


# speedup vs baseline: 301.4730x; 301.4730x over previous
import functools
import math

import jax
import jax.numpy as jnp
import numpy as np
from jax import lax
from jax.experimental import pallas as pl
from jax.experimental.pallas import tpu as pltpu
from jax.experimental.pallas import tpu_sc as plsc

S = 256
B = 4
DIM = 256
HEADS = 8
HD = DIM // HEADS
NN = S + 1
NP = 264
VOCAB_W = 10000
WDIM = 300
CDIM = 50
NTOK = 32
NCHAR = 16
NIDX = B * NTOK
NIDX_PAD = 256


def _edge_count_matrix() -> np.ndarray:
    te = [(i, i + 1) for i in range(S - 1)] + [(i + 1, i) for i in range(S - 1)]
    h2 = [(i, i + 2) for i in range(S - 2)] + [(i + 2, i) for i in range(S - 2)]
    qg = [(0, i + 1) for i in range(S)] + [(i + 1, 0) for i in range(S)]
    iu, ju = np.triu_indices(S, k=1)
    sem = list(zip(iu.tolist(), ju.tolist())) + list(zip(ju.tolist(), iu.tolist()))
    top = [0] + list(range(1, S + 1))
    bot = list(range(1, S + 1)) + [0]
    qry = list(zip(top, bot))
    allp = te + h2 + qg + sem + qry
    src = np.array([p[0] for p in allp])
    dst = np.array([p[1] for p in allp])
    cnt = np.zeros((NP, NP), np.float32)
    np.add.at(cnt, (dst, src), 1.0)
    return cnt


_COUNTS = _edge_count_matrix()


_SC_CORES = 2
_SC_SUBCORES = 16


@functools.lru_cache(maxsize=1)
def _make_word_gather():
    nw = _SC_CORES * _SC_SUBCORES
    rows_per_w = NIDX_PAD // nw
    mesh = plsc.VectorSubcoreMesh(core_axis_name="c", subcore_axis_name="s")

    @functools.partial(
        pl.kernel,
        mesh=mesh,
        compiler_params=pltpu.CompilerParams(use_tc_tiling_on_sc=False),
        out_type=jax.ShapeDtypeStruct((NIDX_PAD, WDIM), jnp.float32),
        scratch_types=[
            pltpu.VMEM((rows_per_w,), jnp.int32),
            pltpu.VMEM((rows_per_w, WDIM), jnp.float32),
            pltpu.SemaphoreType.DMA,
        ],
    )
    def word_gather(table_hbm, idx_hbm, out_hbm, idx_v, rows_v, sem):
        wid = lax.axis_index("s") * _SC_CORES + lax.axis_index("c")
        base = wid * rows_per_w
        pltpu.sync_copy(idx_hbm.at[pl.ds(base, rows_per_w)], idx_v)
        pltpu.async_copy(table_hbm.at[idx_v], rows_v, sem).wait()
        pltpu.sync_copy(rows_v, out_hbm.at[pl.ds(base, rows_per_w)])

    return word_gather


def _fused_body(vid_ref, vmask_ref, qmask_ref, cids_ref, we_ref,
                wv_ref, bv_ref, wenc_ref, benc_ref,
                wembw_ref, wembc_ref, bemb_ref, cemb_ref,
                wq_ref, wk_ref, wvv_ref, wskip_ref, wse_ref, bse_ref,
                cnt_ref, out_ref):
    f32 = jnp.float32
    vf = jnp.dot(vid_ref[0], wv_ref[...], preferred_element_type=f32) + bv_ref[...]
    vf = jnp.dot(vf, wenc_ref[...], preferred_element_type=f32) + benc_ref[...]
    vf = jnp.maximum(vf, 0.0) * vmask_ref[0]

    cids = cids_ref[0]
    ce = None
    cemb = cemb_ref[...]
    lanes = lax.broadcasted_iota(jnp.int32, (NTOK, 128), 1)
    for j in range(NCHAR):
        oh = (cids[:, j:j + 1] == lanes).astype(f32)
        cj = jnp.dot(oh, cemb, preferred_element_type=f32)
        ce = cj if ce is None else jnp.maximum(ce, cj)

    qf = (jnp.dot(we_ref[0], wembw_ref[...], preferred_element_type=f32)
          + jnp.dot(ce, wembc_ref[...], preferred_element_type=f32)
          + bemb_ref[...])
    qf = jnp.dot(qf, wenc_ref[...], preferred_element_type=f32) + benc_ref[...]
    qf = jnp.maximum(qf, 0.0) * qmask_ref[0]
    qnode = jnp.sum(qf, axis=0, keepdims=True) * (1.0 / NTOK)

    x = jnp.concatenate([qnode, vf, jnp.zeros((NP - NN, DIM), f32)], axis=0)

    q_all = jnp.dot(x, wq_ref[...], preferred_element_type=f32)
    k_all = jnp.dot(x, wk_ref[...], preferred_element_type=f32)
    v_all = jnp.dot(x, wvv_ref[...], preferred_element_type=f32)
    skip = jnp.dot(x, wskip_ref[...], preferred_element_type=f32)

    cnt = cnt_ref[...]
    present = cnt > 0.0
    scale = 1.0 / math.sqrt(HD)
    heads = []
    for h in range(HEADS):
        sl = slice(h * HD, (h + 1) * HD)
        logits = lax.dot_general(q_all[:, sl], k_all[:, sl],
                                 (((1,), (1,)), ((), ())),
                                 preferred_element_type=f32) * scale
        lm = jnp.where(present, logits, -1e30)
        m = jnp.max(lm, axis=1, keepdims=True)
        e = jnp.exp(lm - m) * cnt
        denom = jnp.sum(e, axis=1, keepdims=True)
        agg = jnp.dot(e, v_all[:, sl], preferred_element_type=f32)
        heads.append(agg / (denom + 1e-16))
    out = jnp.concatenate(heads, axis=1) + skip

    out_ref[0] = jnp.dot(out, wse_ref[...], preferred_element_type=f32) + bse_ref[...]


def _full(shape):
    return pl.BlockSpec(shape, lambda b: tuple(0 for _ in shape))


_TC_IN_SPECS = [
    pl.BlockSpec((1, S, 1024), lambda b: (b, 0, 0)),
    pl.BlockSpec((1, S, 1), lambda b: (b, 0, 0)),
    pl.BlockSpec((1, NTOK, 1), lambda b: (b, 0, 0)),
    pl.BlockSpec((1, NTOK, NCHAR), lambda b: (b, 0, 0)),
    pl.BlockSpec((1, NTOK, WDIM), lambda b: (b, 0, 0)),
    _full((1024, DIM)), _full((1, DIM)),
    _full((DIM, DIM)), _full((1, DIM)),
    _full((WDIM, DIM)), _full((CDIM, DIM)), _full((1, DIM)),
    _full((128, CDIM)),
    _full((DIM, DIM)), _full((DIM, DIM)), _full((DIM, DIM)), _full((DIM, DIM)),
    _full((DIM, 2)), _full((1, 2)),
    _full((NP, NP)),
]


def kernel(word_ids, char_ids, video_features, v_mask, q_mask, word_emb,
           char_emb, W_emb, b_emb, W_v, b_v, W_enc, b_enc, Wq, Wk, Wv, Wskip,
           W_start, b_start, W_end, b_end):
    f32 = jnp.float32
    idx = word_ids.reshape(-1).astype(jnp.int32)
    idx = jnp.concatenate([idx, idx])
    we = _make_word_gather()(word_emb.astype(f32), idx)[:NIDX].reshape(B, NTOK, WDIM)

    cemb_pad = jnp.zeros((128, CDIM), f32).at[:100].set(char_emb.astype(f32))
    wse = jnp.stack([W_start, W_end], axis=1)
    bse = jnp.stack([b_start, b_end]).reshape(1, 2)

    se = pl.pallas_call(
        _fused_body,
        grid=(B,),
        in_specs=_TC_IN_SPECS,
        out_specs=pl.BlockSpec((1, NP, 2), lambda b: (b, 0, 0)),
        out_shape=jax.ShapeDtypeStruct((B, NP, 2), f32),
    )(video_features, v_mask[..., None], q_mask[..., None],
      char_ids.astype(jnp.int32), we,
      W_v, b_v.reshape(1, DIM), W_enc, b_enc.reshape(1, DIM),
      W_emb[:WDIM], W_emb[WDIM:], b_emb.reshape(1, DIM), cemb_pad,
      Wq, Wk, Wv, Wskip, wse, bse, jnp.asarray(_COUNTS))

    return se[:, 1:NN, 0], se[:, 1:NN, 1]


# speedup vs baseline: 618.6494x; 2.0521x over previous
import functools
import math

import jax
import jax.numpy as jnp
import numpy as np
from jax import lax
from jax.experimental import pallas as pl
from jax.experimental.pallas import tpu as pltpu
from jax.experimental.pallas import tpu_sc as plsc

S = 256
B = 4
DIM = 256
HEADS = 8
HD = DIM // HEADS
NN = S + 1
NP = 264
VOCAB_W = 10000
WDIM = 300
CDIM = 50
NTOK = 32
NCHAR = 16
NIDX = B * NTOK
NIDX_PAD = 256


def _edge_count_matrix() -> np.ndarray:
    te = [(i, i + 1) for i in range(S - 1)] + [(i + 1, i) for i in range(S - 1)]
    h2 = [(i, i + 2) for i in range(S - 2)] + [(i + 2, i) for i in range(S - 2)]
    qg = [(0, i + 1) for i in range(S)] + [(i + 1, 0) for i in range(S)]
    iu, ju = np.triu_indices(S, k=1)
    sem = list(zip(iu.tolist(), ju.tolist())) + list(zip(ju.tolist(), iu.tolist()))
    top = [0] + list(range(1, S + 1))
    bot = list(range(1, S + 1)) + [0]
    qry = list(zip(top, bot))
    allp = te + h2 + qg + sem + qry
    src = np.array([p[0] for p in allp])
    dst = np.array([p[1] for p in allp])
    cnt = np.zeros((NP, NP), np.float32)
    np.add.at(cnt, (dst, src), 1.0)
    return cnt


_COUNTS = _edge_count_matrix()


_SC_CORES = 2
_SC_SUBCORES = 16


@functools.lru_cache(maxsize=1)
def _make_word_gather():
    nw = _SC_CORES * _SC_SUBCORES
    rows_per_w = NIDX_PAD // nw
    mesh = plsc.VectorSubcoreMesh(core_axis_name="c", subcore_axis_name="s")

    @functools.partial(
        pl.kernel,
        mesh=mesh,
        out_type=(jax.ShapeDtypeStruct((NIDX_PAD, 256), jnp.float32),
                  jax.ShapeDtypeStruct((NIDX_PAD, 128), jnp.float32)),
        scratch_types=[
            pltpu.VMEM((rows_per_w,), jnp.int32),
            pltpu.VMEM((rows_per_w, 256), jnp.float32),
            pltpu.VMEM((rows_per_w, 128), jnp.float32),
            pltpu.SemaphoreType.DMA,
        ],
    )
    def word_gather(table_hbm, tail_hbm, idx_hbm, out_hbm, out_tail_hbm,
                    idx_v, rows_v, tail_v, sem):
        wid = lax.axis_index("s") * _SC_CORES + lax.axis_index("c")
        base = wid * rows_per_w
        pltpu.sync_copy(idx_hbm.at[pl.ds(base, rows_per_w)], idx_v)
        pltpu.async_copy(table_hbm.at[idx_v, pl.ds(0, 128)],
                         rows_v.at[:, pl.ds(0, 128)], sem).wait()
        pltpu.async_copy(table_hbm.at[idx_v, pl.ds(128, 128)],
                         rows_v.at[:, pl.ds(128, 128)], sem).wait()
        pltpu.async_copy(tail_hbm.at[idx_v], tail_v, sem).wait()
        pltpu.sync_copy(rows_v, out_hbm.at[pl.ds(base, rows_per_w)])
        pltpu.sync_copy(tail_v, out_tail_hbm.at[pl.ds(base, rows_per_w)])

    return word_gather


def _fused_body(vid_ref, vmask_ref, qmask_ref, cids_ref, we_ref, wet_ref,
                wv_ref, bv_ref, wenc_ref, benc_ref,
                wembw_ref, wembt_ref, wembc_ref, bemb_ref, cemb_ref,
                wq_ref, wk_ref, wvv_ref, wskip_ref, wse_ref, bse_ref,
                cnt_ref, out_ref):
    f32 = jnp.float32
    vf = jnp.dot(vid_ref[0], wv_ref[...], preferred_element_type=f32) + bv_ref[...]
    vf = jnp.dot(vf, wenc_ref[...], preferred_element_type=f32) + benc_ref[...]
    vf = jnp.maximum(vf, 0.0) * vmask_ref[0]

    cids = cids_ref[0]
    ce = None
    cemb = cemb_ref[...]
    lanes = lax.broadcasted_iota(jnp.int32, (NTOK, 128), 1)
    for j in range(NCHAR):
        oh = (cids[:, j:j + 1] == lanes).astype(f32)
        cj = jnp.dot(oh, cemb, preferred_element_type=f32)
        ce = cj if ce is None else jnp.maximum(ce, cj)

    qf = (jnp.dot(we_ref[0], wembw_ref[...], preferred_element_type=f32)
          + jnp.dot(wet_ref[0], wembt_ref[...], preferred_element_type=f32)
          + jnp.dot(ce, wembc_ref[...], preferred_element_type=f32)
          + bemb_ref[...])
    qf = jnp.dot(qf, wenc_ref[...], preferred_element_type=f32) + benc_ref[...]
    qf = jnp.maximum(qf, 0.0) * qmask_ref[0]
    qnode = jnp.sum(qf, axis=0, keepdims=True) * (1.0 / NTOK)

    x = jnp.concatenate([qnode, vf, jnp.zeros((NP - NN, DIM), f32)], axis=0)

    q_all = jnp.dot(x, wq_ref[...], preferred_element_type=f32)
    k_all = jnp.dot(x, wk_ref[...], preferred_element_type=f32)
    v_all = jnp.dot(x, wvv_ref[...], preferred_element_type=f32)
    skip = jnp.dot(x, wskip_ref[...], preferred_element_type=f32)

    cnt = cnt_ref[...]
    present = cnt > 0.0
    scale = 1.0 / math.sqrt(HD)
    heads = []
    for h in range(HEADS):
        sl = slice(h * HD, (h + 1) * HD)
        logits = lax.dot_general(q_all[:, sl], k_all[:, sl],
                                 (((1,), (1,)), ((), ())),
                                 preferred_element_type=f32) * scale
        lm = jnp.where(present, logits, -1e30)
        m = jnp.max(lm, axis=1, keepdims=True)
        e = jnp.exp(lm - m) * cnt
        denom = jnp.sum(e, axis=1, keepdims=True)
        agg = jnp.dot(e, v_all[:, sl], preferred_element_type=f32)
        heads.append(agg / (denom + 1e-16))
    out = jnp.concatenate(heads, axis=1) + skip

    out_ref[0] = jnp.dot(out, wse_ref[...], preferred_element_type=f32) + bse_ref[...]


def _full(shape):
    return pl.BlockSpec(shape, lambda b: tuple(0 for _ in shape))


_TC_IN_SPECS = [
    pl.BlockSpec((1, S, 1024), lambda b: (b, 0, 0)),
    pl.BlockSpec((1, S, 1), lambda b: (b, 0, 0)),
    pl.BlockSpec((1, NTOK, 1), lambda b: (b, 0, 0)),
    pl.BlockSpec((1, NTOK, NCHAR), lambda b: (b, 0, 0)),
    pl.BlockSpec((1, NTOK, 256), lambda b: (b, 0, 0)),
    pl.BlockSpec((1, NTOK, 128), lambda b: (b, 0, 0)),
    _full((1024, DIM)), _full((1, DIM)),
    _full((DIM, DIM)), _full((1, DIM)),
    _full((256, DIM)), _full((128, DIM)),
    _full((CDIM, DIM)), _full((1, DIM)),
    _full((128, CDIM)),
    _full((DIM, DIM)), _full((DIM, DIM)), _full((DIM, DIM)), _full((DIM, DIM)),
    _full((DIM, 2)), _full((1, 2)),
    _full((NP, NP)),
]


def kernel(word_ids, char_ids, video_features, v_mask, q_mask, word_emb,
           char_emb, W_emb, b_emb, W_v, b_v, W_enc, b_enc, Wq, Wk, Wv, Wskip,
           W_start, b_start, W_end, b_end):
    f32 = jnp.float32
    idx = word_ids.reshape(-1).astype(jnp.int32)
    idx = jnp.concatenate([idx, idx])
    tail = jnp.pad(word_emb[:, 256:], ((0, 0), (0, 128 - (WDIM - 256))))
    g, gt = _make_word_gather()(word_emb, tail, idx)
    we = g[:NIDX].reshape(B, NTOK, 256)
    wet = gt[:NIDX].reshape(B, NTOK, 128)

    cemb_pad = jnp.zeros((128, CDIM), f32).at[:100].set(char_emb.astype(f32))
    wse = jnp.stack([W_start, W_end], axis=1)
    bse = jnp.stack([b_start, b_end]).reshape(1, 2)

    se = pl.pallas_call(
        _fused_body,
        grid=(B,),
        in_specs=_TC_IN_SPECS,
        out_specs=pl.BlockSpec((1, NP, 2), lambda b: (b, 0, 0)),
        out_shape=jax.ShapeDtypeStruct((B, NP, 2), f32),
    )(video_features, v_mask[..., None], q_mask[..., None],
      char_ids.astype(jnp.int32), we, wet,
      W_v, b_v.reshape(1, DIM), W_enc, b_enc.reshape(1, DIM),
      W_emb[:256], jnp.pad(W_emb[256:WDIM], ((0, 128 - (WDIM - 256)), (0, 0))),
      W_emb[WDIM:], b_emb.reshape(1, DIM), cemb_pad,
      Wq, Wk, Wv, Wskip, wse, bse, jnp.asarray(_COUNTS))

    return se[:, 1:NN, 0], se[:, 1:NN, 1]


# speedup vs baseline: 722.4751x; 1.1678x over previous
import functools
import math

import jax
import jax.numpy as jnp
import numpy as np
from jax import lax
from jax.experimental import pallas as pl
from jax.experimental.pallas import tpu as pltpu
from jax.experimental.pallas import tpu_sc as plsc

S = 256
B = 4
DIM = 256
HEADS = 8
HD = DIM // HEADS
NN = S + 1
NP = 264
WDIM = 300
CDIM = 50
NTOK = 32
NCHAR = 16
NIDX = B * NTOK


def _edge_count_matrix() -> np.ndarray:
    te = [(i, i + 1) for i in range(S - 1)] + [(i + 1, i) for i in range(S - 1)]
    h2 = [(i, i + 2) for i in range(S - 2)] + [(i + 2, i) for i in range(S - 2)]
    qg = [(0, i + 1) for i in range(S)] + [(i + 1, 0) for i in range(S)]
    iu, ju = np.triu_indices(S, k=1)
    sem = list(zip(iu.tolist(), ju.tolist())) + list(zip(ju.tolist(), iu.tolist()))
    top = [0] + list(range(1, S + 1))
    bot = list(range(1, S + 1)) + [0]
    qry = list(zip(top, bot))
    allp = te + h2 + qg + sem + qry
    src = np.array([p[0] for p in allp])
    dst = np.array([p[1] for p in allp])
    cnt = np.zeros((NP, NP), np.float32)
    np.add.at(cnt, (dst, src), 1.0)
    return cnt


_COUNTS = _edge_count_matrix()


_SC_CORES = 2
_SC_SUBCORES = 16


@functools.lru_cache(maxsize=1)
def _make_word_gather():
    nw = _SC_CORES * _SC_SUBCORES
    rows_per_w = NIDX // nw
    mesh = plsc.VectorSubcoreMesh(core_axis_name="c", subcore_axis_name="s")

    @functools.partial(
        pl.kernel,
        mesh=mesh,
        out_type=jax.ShapeDtypeStruct((NIDX, WDIM), jnp.float32),
        scratch_types=[
            pltpu.VMEM((16,), jnp.int32),
            pltpu.VMEM((rows_per_w, WDIM), jnp.float32),
            pltpu.SemaphoreType.DMA,
        ],
    )
    def word_gather(table_hbm, idx_hbm, out_hbm, idx_v, rows_v, sem):
        wid = lax.axis_index("s") * _SC_CORES + lax.axis_index("c")
        base = wid * rows_per_w
        grp = jnp.minimum((base // 8) * 8, NIDX - 16)
        sel = base - grp
        pltpu.sync_copy(idx_hbm.at[pl.ds(grp, 16)], idx_v)
        ids = idx_v[...]
        handles = []
        for r in range(rows_per_w):
            row_id = jnp.where(
                sel == 0, ids[r],
                jnp.where(sel == 4, ids[r + 4],
                          jnp.where(sel == 8, ids[r + 8], ids[r + 12])))
            handles.append(
                pltpu.async_copy(table_hbm.at[row_id], rows_v.at[r], sem))
        for h in handles:
            h.wait()
        pltpu.sync_copy(rows_v, out_hbm.at[pl.ds(base, rows_per_w)])

    return word_gather


def _fused_body(vid_ref, vmask_ref, qmask_ref, cids_ref, we_ref,
                wv_ref, bv_ref, wenc_ref, benc_ref,
                wembw_ref, wembc_ref, bemb_ref, cemb_ref,
                wq_ref, wk_ref, wvv_ref, wskip_ref,
                wst_ref, bst_ref, wen_ref, ben_ref,
                cnt_ref, start_ref, end_ref):
    f32 = jnp.float32
    vf = jnp.dot(vid_ref[0], wv_ref[...], preferred_element_type=f32) + bv_ref[...]
    vf = jnp.dot(vf, wenc_ref[...], preferred_element_type=f32) + benc_ref[...]
    vf = jnp.maximum(vf, 0.0) * vmask_ref[0]

    cids = cids_ref[0]
    ce = None
    cemb = cemb_ref[...]
    lanes = lax.broadcasted_iota(jnp.int32, (NTOK, 100), 1)
    for j in range(NCHAR):
        oh = (cids[:, j:j + 1] == lanes).astype(f32)
        cj = jnp.dot(oh, cemb, preferred_element_type=f32)
        ce = cj if ce is None else jnp.maximum(ce, cj)

    qf = (jnp.dot(we_ref[0], wembw_ref[...], preferred_element_type=f32)
          + jnp.dot(ce, wembc_ref[...], preferred_element_type=f32)
          + bemb_ref[...])
    qf = jnp.dot(qf, wenc_ref[...], preferred_element_type=f32) + benc_ref[...]
    qf = jnp.maximum(qf, 0.0) * qmask_ref[0]
    qnode = jnp.sum(qf, axis=0, keepdims=True) * (1.0 / NTOK)

    x = jnp.concatenate([qnode, vf, jnp.zeros((NP - NN, DIM), f32)], axis=0)

    q_all = jnp.dot(x, wq_ref[...], preferred_element_type=f32)
    k_all = jnp.dot(x, wk_ref[...], preferred_element_type=f32)
    v_all = jnp.dot(x, wvv_ref[...], preferred_element_type=f32)
    skip = jnp.dot(x, wskip_ref[...], preferred_element_type=f32)

    cnt = cnt_ref[...]
    present = cnt > 0.0
    scale = 1.0 / math.sqrt(HD)
    heads = []
    for h in range(HEADS):
        sl = slice(h * HD, (h + 1) * HD)
        logits = lax.dot_general(q_all[:, sl], k_all[:, sl],
                                 (((1,), (1,)), ((), ())),
                                 preferred_element_type=f32) * scale
        lm = jnp.where(present, logits, -1e30)
        m = jnp.max(lm, axis=1, keepdims=True)
        e = jnp.exp(lm - m) * cnt
        denom = jnp.sum(e, axis=1, keepdims=True)
        agg = jnp.dot(e, v_all[:, sl], preferred_element_type=f32)
        heads.append(agg / (denom + 1e-16))
    out = jnp.concatenate(heads, axis=1) + skip

    st = lax.dot_general(wst_ref[...], out, (((1,), (1,)), ((), ())),
                         preferred_element_type=f32)
    en = lax.dot_general(wen_ref[...], out, (((1,), (1,)), ((), ())),
                         preferred_element_type=f32)
    start_ref[0] = st[:, 1:NN] + bst_ref[...]
    end_ref[0] = en[:, 1:NN] + ben_ref[...]


def _full(shape):
    return pl.BlockSpec(shape, lambda b: tuple(0 for _ in shape))


_TC_IN_SPECS = [
    pl.BlockSpec((1, S, 1024), lambda b: (b, 0, 0)),
    pl.BlockSpec((1, S, 1), lambda b: (b, 0, 0)),
    pl.BlockSpec((1, NTOK, 1), lambda b: (b, 0, 0)),
    pl.BlockSpec((1, NTOK, NCHAR), lambda b: (b, 0, 0)),
    pl.BlockSpec((1, NTOK, WDIM), lambda b: (b, 0, 0)),
    _full((1024, DIM)), _full((1, DIM)),
    _full((DIM, DIM)), _full((1, DIM)),
    _full((WDIM, DIM)), _full((CDIM, DIM)), _full((1, DIM)),
    _full((100, CDIM)),
    _full((DIM, DIM)), _full((DIM, DIM)), _full((DIM, DIM)), _full((DIM, DIM)),
    _full((1, DIM)), _full((1, 1)),
    _full((1, DIM)), _full((1, 1)),
    _full((NP, NP)),
]


def kernel(word_ids, char_ids, video_features, v_mask, q_mask, word_emb,
           char_emb, W_emb, b_emb, W_v, b_v, W_enc, b_enc, Wq, Wk, Wv, Wskip,
           W_start, b_start, W_end, b_end):
    f32 = jnp.float32
    idx = word_ids.reshape(-1).astype(jnp.int32)
    we = _make_word_gather()(word_emb, idx).reshape(B, NTOK, WDIM)

    start, end = pl.pallas_call(
        _fused_body,
        grid=(B,),
        in_specs=_TC_IN_SPECS,
        out_specs=(pl.BlockSpec((1, 1, S), lambda b: (b, 0, 0)),
                   pl.BlockSpec((1, 1, S), lambda b: (b, 0, 0))),
        out_shape=(jax.ShapeDtypeStruct((B, 1, S), f32),
                   jax.ShapeDtypeStruct((B, 1, S), f32)),
    )(video_features, v_mask[..., None], q_mask[..., None],
      char_ids.astype(jnp.int32), we,
      W_v, b_v.reshape(1, DIM), W_enc, b_enc.reshape(1, DIM),
      W_emb[:WDIM], W_emb[WDIM:], b_emb.reshape(1, DIM), char_emb,
      Wq, Wk, Wv, Wskip,
      W_start.reshape(1, DIM), b_start.reshape(1, 1),
      W_end.reshape(1, DIM), b_end.reshape(1, 1),
      jnp.asarray(_COUNTS))
    return start.reshape(B, S), end.reshape(B, S)
